# baseline (speedup 1.0000x reference)
import functools

import jax
import jax.numpy as jnp
from jax import lax
from jax.experimental import pallas as pl
from jax.experimental.pallas import tpu as pltpu
from jax.experimental.pallas import tpu_sc as plsc

N = 10000
E = 160000
NF = 128
H = 3
NEG = 0.2
HALF = H * NF // 2
QF = H * NF // 4

NC = 2
NS = 16
NW = NC * NS

B1 = 800
B3 = 128
NB1 = E // B1
NB3 = E // B3
ROWS_PER_TILE = 624
_ZCOPY = (256, 256, 112)
_TAIL = N - NS * ROWS_PER_TILE


def _tc_a(h_ref, wn_ref, wuv_ref, wv0_ref, wv1_ref, wv2_ref, wv3_ref, aa_ref):
    hb = h_ref[...]
    wv = jnp.dot(hb, wn_ref[...], preferred_element_type=jnp.float32)
    wv0_ref[...] = wv[:, :QF]
    wv1_ref[...] = wv[:, QF:2 * QF]
    wv2_ref[...] = wv[:, 2 * QF:3 * QF]
    wv3_ref[...] = wv[:, 3 * QF:]
    aa_ref[...] = jnp.dot(hb, wuv_ref[...], preferred_element_type=jnp.float32)


def _tc_b(e_ref, wea_ref, ae_ref):
    ae_ref[...] = jnp.dot(e_ref[...], wea_ref[...],
                          preferred_element_type=jnp.float32)


def _tc_c(part_ref, rd_ref):
    s = jnp.sum(part_ref[...], axis=0, keepdims=True)
    rd_ref[...] = jnp.where(s > 0, 1.0 / s, 0.0)


def _tc_d(e_ref, sc_ref, we_ref, s0_ref, s1_ref, s2_ref, s3_ref):
    eb = e_ref[...]
    sc = sc_ref[...]
    bx = sc.shape[0]
    sexp = jnp.broadcast_to(sc[:, :, None], (bx, H, NF)).reshape(bx, H * NF)
    see = jnp.dot(eb, we_ref[...], preferred_element_type=jnp.float32) * sexp
    s0_ref[...] = see[:, :QF]
    s1_ref[...] = see[:, QF:2 * QF]
    s2_ref[...] = see[:, 2 * QF:3 * QF]
    s3_ref[...] = see[:, 3 * QF:]


def _tc_e(a0_ref, a1_ref, a2_ref, a3_ref, ws_ref, b_ref, out_ref):
    acc = b_ref[...]
    for q, a_ref in enumerate((a0_ref, a1_ref, a2_ref, a3_ref)):
        acc = acc + jnp.dot(a_ref[...], ws_ref[pl.ds(q * QF, QF), :],
                            preferred_element_type=jnp.float32)
    out_ref[...] = acc


def _sc1_body(src_hbm, dst_hbm, ae_hbm, aa_hbm, ex_hbm, part_hbm,
              aa_v, acc_v, src_v, dst_v, ae_v, ex_v):
    wid = lax.axis_index("s") * NC + lax.axis_index("c")
    pltpu.sync_copy(aa_hbm, aa_v)

    def zero_body(i, _):
        acc_v[pl.ds(i * 16, 16)] = jnp.zeros((16,), jnp.float32)
        return 0
    lax.fori_loop(0, (N * H) // 16, zero_body, 0)

    def blk_body(b, _):
        base = b * B1
        pltpu.sync_copy(src_hbm.at[pl.ds(base, B1)], src_v)
        pltpu.sync_copy(dst_hbm.at[pl.ds(base, B1)], dst_v)
        pltpu.sync_copy(ae_hbm.at[pl.ds(base * H, B1 * H)], ae_v)

        def vec_body(i, _):
            s16 = src_v[pl.ds(i * 16, 16)]
            d16 = dst_v[pl.ds(i * 16, 16)]
            lane = lax.iota(jnp.int32, 16) + i * 16
            for h in range(H):
                au = plsc.load_gather(aa_v, [s16 * (2 * H) + h])
                av = plsc.load_gather(aa_v, [d16 * (2 * H) + (H + h)])
                ae16 = plsc.load_gather(ae_v, [lane * H + h])
                lg = au + av + ae16
                lg = jnp.where(lg >= 0, lg, lg * NEG)
                ex16 = jnp.exp(lg)
                plsc.store_scatter(ex_v, [lane * H + h], ex16)
                plsc.addupdate_scatter(acc_v, [d16 * H + h], ex16)
            return 0
        lax.fori_loop(0, B1 // 16, vec_body, 0)
        pltpu.sync_copy(ex_v, ex_hbm.at[pl.ds(base * H, B1 * H)])
        return 0

    nblk = (NB1 - wid + NW - 1) // NW
    lax.fori_loop(0, nblk, lambda k, c: blk_body(wid + k * NW, c), 0)
    pltpu.sync_copy(acc_v, part_hbm.at[wid])


def _sc2_body(dst_hbm, ex_hbm, rd_hbm, sc_hbm,
              rd_v, dst_v, ex_v, sc_v):
    wid = lax.axis_index("s") * NC + lax.axis_index("c")
    pltpu.sync_copy(rd_hbm, rd_v)

    def blk_body(b, _):
        base = b * B1
        pltpu.sync_copy(dst_hbm.at[pl.ds(base, B1)], dst_v)
        pltpu.sync_copy(ex_hbm.at[pl.ds(base * H, B1 * H)], ex_v)

        def vec_body(i, _):
            d16 = dst_v[pl.ds(i * 16, 16)]
            lane = lax.iota(jnp.int32, 16) + i * 16
            for h in range(H):
                ex16 = plsc.load_gather(ex_v, [lane * H + h])
                r16 = plsc.load_gather(rd_v, [d16 * H + h])
                plsc.store_scatter(sc_v, [lane * H + h], ex16 * r16)
            return 0
        lax.fori_loop(0, B1 // 16, vec_body, 0)
        pltpu.sync_copy(sc_v, sc_hbm.at[pl.ds(base * H, B1 * H)])
        return 0

    nblk = (NB1 - wid + NW - 1) // NW
    lax.fori_loop(0, nblk, lambda k, c: blk_body(wid + k * NW, c), 0)


def _sc3_body(src_hbm, dst_hbm, wv0_hbm, wv1_hbm, wv2_hbm, wv3_hbm,
              see0_hbm, see1_hbm, see2_hbm, see3_hbm,
              agg0_hbm, agg1_hbm, agg2_hbm, agg3_hbm,
              u0, u1, se0, se1, src0, src1, dst0, dst1,
              sem_u0, sem_u1, sem_s0, sem_s1, agg_s):
    c = lax.axis_index("c")
    s = lax.axis_index("s")
    u_v = (u0, u1)
    se_v = (se0, se1)
    src_v = (src0, src1)
    dst_v = (dst0, dst1)
    sem_u = (sem_u0, sem_u1)
    sem_s = (sem_s0, sem_s1)
    wv_hbm = (wv0_hbm, wv1_hbm, wv2_hbm, wv3_hbm)
    see_hbm = (see0_hbm, see1_hbm, see2_hbm, see3_hbm)
    agg_hbm = (agg0_hbm, agg1_hbm, agg2_hbm, agg3_hbm)
    nblk = (NB3 - s + NS - 1) // NS

    for p in range(2):
        def zbuf_body(i, _):
            for j in range(QF // 16):
                u0[i, pl.ds(j * 16, 16)] = jnp.zeros((16,), jnp.float32)
            return 0
        lax.fori_loop(0, B3, zbuf_body, 0)
        off = 0
        for nrows in _ZCOPY:
            for rep in range(-(-nrows // B3)):
                nn = min(B3, nrows - rep * B3)
                pltpu.sync_copy(
                    u0.at[pl.ds(0, nn)],
                    agg_s.at[pl.ds(s * ROWS_PER_TILE + off + rep * B3, nn)])
            off += nrows

        @pl.when(s == NS - 1)
        def _():
            pltpu.sync_copy(u0.at[pl.ds(0, _TAIL)],
                            agg_s.at[pl.ds(NS * ROWS_PER_TILE, _TAIL)])

        plsc.subcore_barrier()

        def issue(b, j):
            base = (s + j * NS) * B3
            pltpu.sync_copy(src_hbm.at[pl.ds(base, B3)], src_v[b])
            pltpu.sync_copy(dst_hbm.at[pl.ds(base, B3)], dst_v[b])

            @pl.when(c == 0)
            def _():
                pltpu.async_copy(wv_hbm[2 * p].at[src_v[b]], u_v[b], sem_u[b])
                pltpu.async_copy(see_hbm[2 * p].at[pl.ds(base, B3)],
                                 se_v[b], sem_s[b])

            @pl.when(c == 1)
            def _():
                pltpu.async_copy(wv_hbm[2 * p + 1].at[src_v[b]],
                                 u_v[b], sem_u[b])
                pltpu.async_copy(see_hbm[2 * p + 1].at[pl.ds(base, B3)],
                                 se_v[b], sem_s[b])

        for b in range(2):
            @pl.when(b < nblk)
            def _(b=b, issue=issue):
                issue(b, b)

        def k2_body(k2, _, issue=issue):
            for b in range(2):
                j = 2 * k2 + b

                @pl.when(j < nblk)
                def _(b=b, j=j):
                    pltpu.make_async_copy(wv0_hbm.at[src_v[b]], u_v[b],
                                          sem_u[b]).wait()
                    pltpu.make_async_copy(see0_hbm.at[pl.ds(0, B3)], se_v[b],
                                          sem_s[b]).wait()

                    def mul_body(i, _):
                        for jj in range(QF // 16):
                            u_v[b][i, pl.ds(jj * 16, 16)] = (
                                u_v[b][i, pl.ds(jj * 16, 16)]
                                * se_v[b][i, pl.ds(jj * 16, 16)])
                        return 0
                    lax.fori_loop(0, B3, mul_body, 0)

                    pltpu.sync_copy(u_v[b], agg_s.at[dst_v[b]], add=True)

                    @pl.when(j + 2 < nblk)
                    def _():
                        issue(b, j + 2)
            return 0

        lax.fori_loop(0, (nblk + 1) // 2, k2_body, 0)
        plsc.subcore_barrier()

        def _dump(row, nrows, p=p):
            @pl.when(c == 0)
            def _():
                pltpu.sync_copy(agg_s.at[pl.ds(row, nrows)],
                                agg_hbm[2 * p].at[pl.ds(row, nrows)])

            @pl.when(c == 1)
            def _():
                pltpu.sync_copy(agg_s.at[pl.ds(row, nrows)],
                                agg_hbm[2 * p + 1].at[pl.ds(row, nrows)])

        off = 0
        for nrows in _ZCOPY:
            _dump(s * ROWS_PER_TILE + off, nrows)
            off += nrows

        @pl.when(s == NS - 1)
        def _():
            _dump(NS * ROWS_PER_TILE, _TAIL)

        plsc.subcore_barrier()


@jax.jit
def kernel(h, edge_index, e, W_node, W_edge, w_att, W_scale, bias):
    f32 = jnp.float32
    src = edge_index[0].astype(jnp.int32)
    dst = edge_index[1].astype(jnp.int32)

    wa = w_att[0]
    wa_u, wa_e, wa_v = wa[:, :NF], wa[:, NF:2 * NF], wa[:, 2 * NF:]
    wn3 = W_node.reshape(NF, H, NF)
    wu = jnp.einsum("fhg,hg->fh", wn3, wa_u)
    wv_fold = jnp.einsum("fhg,hg->fh", wn3, wa_v)
    w_uv = jnp.concatenate([wu, wv_fold], axis=1)
    we3 = W_edge.reshape(e.shape[1], H, NF)
    wea = jnp.einsum("fhg,hg->fh", we3, wa_e)

    BN = 1000
    wvq = pl.pallas_call(
        _tc_a,
        grid=(N // BN,),
        in_specs=[
            pl.BlockSpec((BN, NF), lambda i: (i, 0)),
            pl.BlockSpec((NF, H * NF), lambda i: (0, 0)),
            pl.BlockSpec((NF, 2 * H), lambda i: (0, 0)),
        ],
        out_specs=[pl.BlockSpec((BN, QF), lambda i: (i, 0))] * 4
        + [pl.BlockSpec((BN, 2 * H), lambda i: (i, 0))],
        out_shape=[jax.ShapeDtypeStruct((N, QF), f32)] * 4
        + [jax.ShapeDtypeStruct((N, 2 * H), f32)],
    )(h, W_node, w_uv)
    wv0, wv1, wv2, wv3, aa = wvq

    BE = 2000
    ae = pl.pallas_call(
        _tc_b,
        grid=(E // BE,),
        in_specs=[
            pl.BlockSpec((BE, e.shape[1]), lambda i: (i, 0)),
            pl.BlockSpec((e.shape[1], H), lambda i: (0, 0)),
        ],
        out_specs=pl.BlockSpec((BE, H), lambda i: (i, 0)),
        out_shape=jax.ShapeDtypeStruct((E, H), f32),
    )(e, wea)

    mesh = plsc.VectorSubcoreMesh(core_axis_name="c", subcore_axis_name="s")
    sc_params = pltpu.CompilerParams(needs_layout_passes=False,
                                     use_tc_tiling_on_sc=False)

    sc1 = pl.kernel(
        _sc1_body,
        out_type=[
            jax.ShapeDtypeStruct((E * H,), f32),
            jax.ShapeDtypeStruct((NW, N * H), f32),
        ],
        mesh=mesh,
        scratch_types=[
            pltpu.VMEM((N * 2 * H,), f32),
            pltpu.VMEM((N * H,), f32),
            pltpu.VMEM((B1,), jnp.int32),
            pltpu.VMEM((B1,), jnp.int32),
            pltpu.VMEM((B1 * H,), f32),
            pltpu.VMEM((B1 * H,), f32),
        ],
        compiler_params=sc_params,
    )
    ex, part = sc1(src, dst, ae.reshape(E * H), aa.reshape(N * 2 * H))

    rd = pl.pallas_call(
        _tc_c,
        grid=(1,),
        in_specs=[pl.BlockSpec((NW, N * H), lambda i: (0, 0))],
        out_specs=pl.BlockSpec((1, N * H), lambda i: (0, 0)),
        out_shape=jax.ShapeDtypeStruct((1, N * H), f32),
    )(part)

    sc2 = pl.kernel(
        _sc2_body,
        out_type=jax.ShapeDtypeStruct((E * H,), f32),
        mesh=mesh,
        scratch_types=[
            pltpu.VMEM((N * H,), f32),
            pltpu.VMEM((B1,), jnp.int32),
            pltpu.VMEM((B1 * H,), f32),
            pltpu.VMEM((B1 * H,), f32),
        ],
        compiler_params=sc_params,
    )
    score = sc2(dst, ex, rd.reshape(N * H))

    BD = 1000
    see_q = pl.pallas_call(
        _tc_d,
        grid=(E // BD,),
        in_specs=[
            pl.BlockSpec((BD, e.shape[1]), lambda i: (i, 0)),
            pl.BlockSpec((BD, H), lambda i: (i, 0)),
            pl.BlockSpec((e.shape[1], H * NF), lambda i: (0, 0)),
        ],
        out_specs=[pl.BlockSpec((BD, QF), lambda i: (i, 0))] * 4,
        out_shape=[jax.ShapeDtypeStruct((E, QF), f32)] * 4,
    )(e, score.reshape(E, H), W_edge)

    sc3 = pl.kernel(
        _sc3_body,
        out_type=[jax.ShapeDtypeStruct((N, QF), f32)] * 4,
        mesh=mesh,
        scratch_types=(
            [pltpu.VMEM((B3, QF), f32)] * 4
            + [pltpu.VMEM((B3,), jnp.int32)] * 4
            + [pltpu.SemaphoreType.DMA] * 4
            + [pltpu.VMEM_SHARED((N, QF), f32)]
        ),
        compiler_params=sc_params,
    )
    agg0, agg1, agg2, agg3 = sc3(src, dst, wv0, wv1, wv2, wv3,
                                 see_q[0], see_q[1], see_q[2], see_q[3])

    out = pl.pallas_call(
        _tc_e,
        grid=(N // BN,),
        in_specs=[pl.BlockSpec((BN, QF), lambda i: (i, 0))] * 4
        + [
            pl.BlockSpec((H * NF, NF), lambda i: (0, 0)),
            pl.BlockSpec((1, NF), lambda i: (0, 0)),
        ],
        out_specs=pl.BlockSpec((BN, NF), lambda i: (i, 0)),
        out_shape=jax.ShapeDtypeStruct((N, NF), f32),
    )(agg0, agg1, agg2, agg3, W_scale, bias.reshape(1, NF))

    return out


# speedup vs baseline: 1.1389x; 1.1389x over previous
import functools

import jax
import jax.numpy as jnp
from jax import lax
from jax.experimental import pallas as pl
from jax.experimental.pallas import tpu as pltpu
from jax.experimental.pallas import tpu_sc as plsc

N = 10000
E = 160000
NF = 128
H = 3
NEG = 0.2
HALF = H * NF // 2
QF = H * NF // 4

NC = 2
NS = 16
NW = NC * NS

B1 = 800
B3 = 160
NB1 = E // B1
NB3 = E // B3
ROWS_PER_TILE = 624
_ZCOPY = (256, 256, 112)
_TAIL = N - NS * ROWS_PER_TILE


def _tc_a(h_ref, wn_ref, wuv_ref, wv0_ref, wv1_ref, wv2_ref, wv3_ref, aa_ref):
    hb = h_ref[...]
    wv = jnp.dot(hb, wn_ref[...], preferred_element_type=jnp.float32)
    wv0_ref[...] = wv[:, :QF]
    wv1_ref[...] = wv[:, QF:2 * QF]
    wv2_ref[...] = wv[:, 2 * QF:3 * QF]
    wv3_ref[...] = wv[:, 3 * QF:]
    aa_ref[...] = jnp.dot(hb, wuv_ref[...], preferred_element_type=jnp.float32)


def _tc_b(e_ref, wea_ref, ae_ref):
    ae_ref[...] = jnp.dot(e_ref[...], wea_ref[...],
                          preferred_element_type=jnp.float32)


def _tc_c(part_ref, rd_ref):
    s = jnp.sum(part_ref[...], axis=0, keepdims=True)
    rd_ref[...] = jnp.where(s > 0, 1.0 / s, 0.0)


def _tc_d(e_ref, sc_ref, we_ref, s0_ref, s1_ref, s2_ref, s3_ref):
    eb = e_ref[...]
    sc = sc_ref[...]
    bx = sc.shape[0]
    sexp = jnp.broadcast_to(sc[:, :, None], (bx, H, NF)).reshape(bx, H * NF)
    see = jnp.dot(eb, we_ref[...], preferred_element_type=jnp.float32) * sexp
    s0_ref[...] = see[:, :QF]
    s1_ref[...] = see[:, QF:2 * QF]
    s2_ref[...] = see[:, 2 * QF:3 * QF]
    s3_ref[...] = see[:, 3 * QF:]


def _tc_e(a0_ref, a1_ref, a2_ref, a3_ref, ws_ref, b_ref, out_ref):
    acc = b_ref[...]
    for q, a_ref in enumerate((a0_ref, a1_ref, a2_ref, a3_ref)):
        acc = acc + jnp.dot(a_ref[...], ws_ref[pl.ds(q * QF, QF), :],
                            preferred_element_type=jnp.float32)
    out_ref[...] = acc


def _sc1_body(src_hbm, dst_hbm, ae_hbm, aa_hbm, ex_hbm, part_hbm,
              aa_v, acc_v, src_v, dst_v, ae_v, ex_v):
    wid = lax.axis_index("s") * NC + lax.axis_index("c")
    pltpu.sync_copy(aa_hbm, aa_v)

    def zero_body(i, _):
        acc_v[pl.ds(i * 16, 16)] = jnp.zeros((16,), jnp.float32)
        return 0
    lax.fori_loop(0, (N * H) // 16, zero_body, 0)

    def blk_body(b, _):
        base = b * B1
        pltpu.sync_copy(src_hbm.at[pl.ds(base, B1)], src_v)
        pltpu.sync_copy(dst_hbm.at[pl.ds(base, B1)], dst_v)
        pltpu.sync_copy(ae_hbm.at[pl.ds(base * H, B1 * H)], ae_v)

        def vec_body(i, _):
            s16 = src_v[pl.ds(i * 16, 16)]
            d16 = dst_v[pl.ds(i * 16, 16)]
            lane = lax.iota(jnp.int32, 16) + i * 16
            for h in range(H):
                au = plsc.load_gather(aa_v, [s16 * (2 * H) + h])
                av = plsc.load_gather(aa_v, [d16 * (2 * H) + (H + h)])
                ae16 = plsc.load_gather(ae_v, [lane * H + h])
                lg = au + av + ae16
                lg = jnp.where(lg >= 0, lg, lg * NEG)
                ex16 = jnp.exp(lg)
                plsc.store_scatter(ex_v, [lane * H + h], ex16)
                plsc.addupdate_scatter(acc_v, [d16 * H + h], ex16)
            return 0
        lax.fori_loop(0, B1 // 16, vec_body, 0)
        pltpu.sync_copy(ex_v, ex_hbm.at[pl.ds(base * H, B1 * H)])
        return 0

    nblk = (NB1 - wid + NW - 1) // NW
    lax.fori_loop(0, nblk, lambda k, c: blk_body(wid + k * NW, c), 0)
    pltpu.sync_copy(acc_v, part_hbm.at[wid])


def _sc2_body(dst_hbm, ex_hbm, rd_hbm, sc_hbm,
              rd_v, dst_v, ex_v, sc_v):
    wid = lax.axis_index("s") * NC + lax.axis_index("c")
    pltpu.sync_copy(rd_hbm, rd_v)

    def blk_body(b, _):
        base = b * B1
        pltpu.sync_copy(dst_hbm.at[pl.ds(base, B1)], dst_v)
        pltpu.sync_copy(ex_hbm.at[pl.ds(base * H, B1 * H)], ex_v)

        def vec_body(i, _):
            d16 = dst_v[pl.ds(i * 16, 16)]
            lane = lax.iota(jnp.int32, 16) + i * 16
            for h in range(H):
                ex16 = plsc.load_gather(ex_v, [lane * H + h])
                r16 = plsc.load_gather(rd_v, [d16 * H + h])
                plsc.store_scatter(sc_v, [lane * H + h], ex16 * r16)
            return 0
        lax.fori_loop(0, B1 // 16, vec_body, 0)
        pltpu.sync_copy(sc_v, sc_hbm.at[pl.ds(base * H, B1 * H)])
        return 0

    nblk = (NB1 - wid + NW - 1) // NW
    lax.fori_loop(0, nblk, lambda k, c: blk_body(wid + k * NW, c), 0)


def _sc3_body(src_hbm, dst_hbm, wva_hbm, wvb_hbm, seea_hbm, seeb_hbm,
              agga_hbm, aggb_hbm,
              u0, u1, se0, se1, src0, src1, dst0, dst1,
              sem_u0, sem_u1, sem_s0, sem_s1, agg_s):
    c = lax.axis_index("c")
    s = lax.axis_index("s")
    u_v = (u0, u1)
    se_v = (se0, se1)
    src_v = (src0, src1)
    dst_v = (dst0, dst1)
    sem_u = (sem_u0, sem_u1)
    sem_s = (sem_s0, sem_s1)

    def zbuf_body(i, _):
        for j in range(QF // 16):
            u0[i, pl.ds(j * 16, 16)] = jnp.zeros((16,), jnp.float32)
        return 0
    lax.fori_loop(0, B3, zbuf_body, 0)
    off = 0
    for nrows in _ZCOPY:
        for rep in range(-(-nrows // B3)):
            nn = min(B3, nrows - rep * B3)
            pltpu.sync_copy(
                u0.at[pl.ds(0, nn)],
                agg_s.at[pl.ds(s * ROWS_PER_TILE + off + rep * B3, nn)])
        off += nrows

    @pl.when(s == NS - 1)
    def _():
        pltpu.sync_copy(u0.at[pl.ds(0, _TAIL)],
                        agg_s.at[pl.ds(NS * ROWS_PER_TILE, _TAIL)])

    plsc.subcore_barrier()

    nblk = (NB3 - s + NS - 1) // NS

    def issue(b, j):
        base = (s + j * NS) * B3
        pltpu.sync_copy(src_hbm.at[pl.ds(base, B3)], src_v[b])
        pltpu.sync_copy(dst_hbm.at[pl.ds(base, B3)], dst_v[b])

        @pl.when(c == 0)
        def _():
            pltpu.async_copy(wva_hbm.at[src_v[b]], u_v[b], sem_u[b])
            pltpu.async_copy(seea_hbm.at[pl.ds(base, B3)], se_v[b], sem_s[b])

        @pl.when(c == 1)
        def _():
            pltpu.async_copy(wvb_hbm.at[src_v[b]], u_v[b], sem_u[b])
            pltpu.async_copy(seeb_hbm.at[pl.ds(base, B3)], se_v[b], sem_s[b])

    for b in range(2):
        @pl.when(b < nblk)
        def _(b=b):
            issue(b, b)

    def k2_body(k2, _):
        for b in range(2):
            j = 2 * k2 + b

            @pl.when(j < nblk)
            def _(b=b, j=j):
                pltpu.make_async_copy(wva_hbm.at[src_v[b]], u_v[b],
                                      sem_u[b]).wait()
                pltpu.make_async_copy(seea_hbm.at[pl.ds(0, B3)], se_v[b],
                                      sem_s[b]).wait()

                def mul_body(i, _):
                    for jj in range(QF // 16):
                        u_v[b][i, pl.ds(jj * 16, 16)] = (
                            u_v[b][i, pl.ds(jj * 16, 16)]
                            * se_v[b][i, pl.ds(jj * 16, 16)])
                    return 0
                lax.fori_loop(0, B3, mul_body, 0)

                pltpu.sync_copy(u_v[b], agg_s.at[dst_v[b]], add=True)

                @pl.when(j + 2 < nblk)
                def _():
                    issue(b, j + 2)
        return 0

    lax.fori_loop(0, (nblk + 1) // 2, k2_body, 0)
    plsc.subcore_barrier()

    def _dump(row, nrows):
        @pl.when(c == 0)
        def _():
            pltpu.sync_copy(agg_s.at[pl.ds(row, nrows)],
                            agga_hbm.at[pl.ds(row, nrows)])

        @pl.when(c == 1)
        def _():
            pltpu.sync_copy(agg_s.at[pl.ds(row, nrows)],
                            aggb_hbm.at[pl.ds(row, nrows)])

    off = 0
    for nrows in _ZCOPY:
        _dump(s * ROWS_PER_TILE + off, nrows)
        off += nrows

    @pl.when(s == NS - 1)
    def _():
        _dump(NS * ROWS_PER_TILE, _TAIL)


@jax.jit
def kernel(h, edge_index, e, W_node, W_edge, w_att, W_scale, bias):
    f32 = jnp.float32
    src = edge_index[0].astype(jnp.int32)
    dst = edge_index[1].astype(jnp.int32)

    wa = w_att[0]
    wa_u, wa_e, wa_v = wa[:, :NF], wa[:, NF:2 * NF], wa[:, 2 * NF:]
    wn3 = W_node.reshape(NF, H, NF)
    wu = jnp.einsum("fhg,hg->fh", wn3, wa_u)
    wv_fold = jnp.einsum("fhg,hg->fh", wn3, wa_v)
    w_uv = jnp.concatenate([wu, wv_fold], axis=1)
    we3 = W_edge.reshape(e.shape[1], H, NF)
    wea = jnp.einsum("fhg,hg->fh", we3, wa_e)

    BN = 1000
    wvq = pl.pallas_call(
        _tc_a,
        grid=(N // BN,),
        in_specs=[
            pl.BlockSpec((BN, NF), lambda i: (i, 0)),
            pl.BlockSpec((NF, H * NF), lambda i: (0, 0)),
            pl.BlockSpec((NF, 2 * H), lambda i: (0, 0)),
        ],
        out_specs=[pl.BlockSpec((BN, QF), lambda i: (i, 0))] * 4
        + [pl.BlockSpec((BN, 2 * H), lambda i: (i, 0))],
        out_shape=[jax.ShapeDtypeStruct((N, QF), f32)] * 4
        + [jax.ShapeDtypeStruct((N, 2 * H), f32)],
    )(h, W_node, w_uv)
    wv0, wv1, wv2, wv3, aa = wvq

    BE = 2000
    ae = pl.pallas_call(
        _tc_b,
        grid=(E // BE,),
        in_specs=[
            pl.BlockSpec((BE, e.shape[1]), lambda i: (i, 0)),
            pl.BlockSpec((e.shape[1], H), lambda i: (0, 0)),
        ],
        out_specs=pl.BlockSpec((BE, H), lambda i: (i, 0)),
        out_shape=jax.ShapeDtypeStruct((E, H), f32),
    )(e, wea)

    mesh = plsc.VectorSubcoreMesh(core_axis_name="c", subcore_axis_name="s")
    sc_params = pltpu.CompilerParams(needs_layout_passes=False,
                                     use_tc_tiling_on_sc=False)

    sc1 = pl.kernel(
        _sc1_body,
        out_type=[
            jax.ShapeDtypeStruct((E * H,), f32),
            jax.ShapeDtypeStruct((NW, N * H), f32),
        ],
        mesh=mesh,
        scratch_types=[
            pltpu.VMEM((N * 2 * H,), f32),
            pltpu.VMEM((N * H,), f32),
            pltpu.VMEM((B1,), jnp.int32),
            pltpu.VMEM((B1,), jnp.int32),
            pltpu.VMEM((B1 * H,), f32),
            pltpu.VMEM((B1 * H,), f32),
        ],
        compiler_params=sc_params,
    )
    ex, part = sc1(src, dst, ae.reshape(E * H), aa.reshape(N * 2 * H))

    rd = pl.pallas_call(
        _tc_c,
        grid=(1,),
        in_specs=[pl.BlockSpec((NW, N * H), lambda i: (0, 0))],
        out_specs=pl.BlockSpec((1, N * H), lambda i: (0, 0)),
        out_shape=jax.ShapeDtypeStruct((1, N * H), f32),
    )(part)

    sc2 = pl.kernel(
        _sc2_body,
        out_type=jax.ShapeDtypeStruct((E * H,), f32),
        mesh=mesh,
        scratch_types=[
            pltpu.VMEM((N * H,), f32),
            pltpu.VMEM((B1,), jnp.int32),
            pltpu.VMEM((B1 * H,), f32),
            pltpu.VMEM((B1 * H,), f32),
        ],
        compiler_params=sc_params,
    )
    score = sc2(dst, ex, rd.reshape(N * H))

    BD = 1000
    see_q = pl.pallas_call(
        _tc_d,
        grid=(E // BD,),
        in_specs=[
            pl.BlockSpec((BD, e.shape[1]), lambda i: (i, 0)),
            pl.BlockSpec((BD, H), lambda i: (i, 0)),
            pl.BlockSpec((e.shape[1], H * NF), lambda i: (0, 0)),
        ],
        out_specs=[pl.BlockSpec((BD, QF), lambda i: (i, 0))] * 4,
        out_shape=[jax.ShapeDtypeStruct((E, QF), f32)] * 4,
    )(e, score.reshape(E, H), W_edge)

    sc3 = pl.kernel(
        _sc3_body,
        out_type=[jax.ShapeDtypeStruct((N, QF), f32)] * 2,
        mesh=mesh,
        scratch_types=(
            [pltpu.VMEM((B3, QF), f32)] * 4
            + [pltpu.VMEM((B3,), jnp.int32)] * 4
            + [pltpu.SemaphoreType.DMA] * 4
            + [pltpu.VMEM_SHARED((N, QF), f32)]
        ),
        compiler_params=sc_params,
    )
    agg0, agg1 = sc3(src, dst, wv0, wv1, see_q[0], see_q[1])
    agg2, agg3 = sc3(src, dst, wv2, wv3, see_q[2], see_q[3])

    out = pl.pallas_call(
        _tc_e,
        grid=(N // BN,),
        in_specs=[pl.BlockSpec((BN, QF), lambda i: (i, 0))] * 4
        + [
            pl.BlockSpec((H * NF, NF), lambda i: (0, 0)),
            pl.BlockSpec((1, NF), lambda i: (0, 0)),
        ],
        out_specs=pl.BlockSpec((BN, NF), lambda i: (i, 0)),
        out_shape=jax.ShapeDtypeStruct((N, NF), f32),
    )(agg0, agg1, agg2, agg3, W_scale, bias.reshape(1, NF))

    return out


# speedup vs baseline: 1.1441x; 1.0045x over previous
import functools

import jax
import jax.numpy as jnp
from jax import lax
from jax.experimental import pallas as pl
from jax.experimental.pallas import tpu as pltpu
from jax.experimental.pallas import tpu_sc as plsc

N = 10000
E = 160000
NF = 128
H = 3
NEG = 0.2
HALF = H * NF // 2
QF = H * NF // 4

NC = 2
NS = 16
NW = NC * NS

B1 = 800
B3 = 160
NB1 = E // B1
NB3 = E // B3
ROWS_PER_TILE = 624
_ZCOPY = (256, 256, 112)
_TAIL = N - NS * ROWS_PER_TILE


def _tc_a1(h_ref, wuv_ref, aa_ref):
    aa_ref[...] = jnp.dot(h_ref[...], wuv_ref[...],
                          preferred_element_type=jnp.float32)


def _tc_a2(h_ref, wn_ref, wv0_ref, wv1_ref, wv2_ref, wv3_ref):
    wv = jnp.dot(h_ref[...], wn_ref[...], preferred_element_type=jnp.float32)
    wv0_ref[...] = wv[:, :QF]
    wv1_ref[...] = wv[:, QF:2 * QF]
    wv2_ref[...] = wv[:, 2 * QF:3 * QF]
    wv3_ref[...] = wv[:, 3 * QF:]


def _tc_b(e_ref, wea_ref, ae_ref):
    ae_ref[...] = jnp.dot(e_ref[...], wea_ref[...],
                          preferred_element_type=jnp.float32)


def _tc_c(part_ref, rd_ref):
    s = jnp.sum(part_ref[...], axis=0, keepdims=True)
    rd_ref[...] = jnp.where(s > 0, 1.0 / s, 0.0)


def _tc_d(e_ref, sc_ref, we_ref, s0_ref, s1_ref, s2_ref, s3_ref):
    eb = e_ref[...]
    sc = sc_ref[...]
    bx = sc.shape[0]
    sexp = jnp.broadcast_to(sc[:, :, None], (bx, H, NF)).reshape(bx, H * NF)
    see = jnp.dot(eb, we_ref[...], preferred_element_type=jnp.float32) * sexp
    s0_ref[...] = see[:, :QF]
    s1_ref[...] = see[:, QF:2 * QF]
    s2_ref[...] = see[:, 2 * QF:3 * QF]
    s3_ref[...] = see[:, 3 * QF:]


def _tc_e(a0_ref, a1_ref, a2_ref, a3_ref, ws_ref, b_ref, out_ref):
    acc = b_ref[...]
    for q, a_ref in enumerate((a0_ref, a1_ref, a2_ref, a3_ref)):
        acc = acc + jnp.dot(a_ref[...], ws_ref[pl.ds(q * QF, QF), :],
                            preferred_element_type=jnp.float32)
    out_ref[...] = acc


def _sc1_body(src_hbm, dst_hbm, ae_hbm, aa_hbm, ex_hbm, part_hbm,
              aa_v, acc_v, src_v, dst_v, ae_v, ex_v):
    wid = lax.axis_index("s") * NC + lax.axis_index("c")
    pltpu.sync_copy(aa_hbm, aa_v)

    def zero_body(i, _):
        acc_v[pl.ds(i * 16, 16)] = jnp.zeros((16,), jnp.float32)
        return 0
    lax.fori_loop(0, (N * H) // 16, zero_body, 0)

    def blk_body(b, _):
        base = b * B1
        pltpu.sync_copy(src_hbm.at[pl.ds(base, B1)], src_v)
        pltpu.sync_copy(dst_hbm.at[pl.ds(base, B1)], dst_v)
        pltpu.sync_copy(ae_hbm.at[pl.ds(base * H, B1 * H)], ae_v)

        def vec_body(i, _):
            s16 = src_v[pl.ds(i * 16, 16)]
            d16 = dst_v[pl.ds(i * 16, 16)]
            lane = lax.iota(jnp.int32, 16) + i * 16
            for h in range(H):
                au = plsc.load_gather(aa_v, [s16 * (2 * H) + h])
                av = plsc.load_gather(aa_v, [d16 * (2 * H) + (H + h)])
                ae16 = plsc.load_gather(ae_v, [lane * H + h])
                lg = au + av + ae16
                lg = jnp.where(lg >= 0, lg, lg * NEG)
                ex16 = jnp.exp(lg)
                plsc.store_scatter(ex_v, [lane * H + h], ex16)
                plsc.addupdate_scatter(acc_v, [d16 * H + h], ex16)
            return 0
        lax.fori_loop(0, B1 // 16, vec_body, 0)
        pltpu.sync_copy(ex_v, ex_hbm.at[pl.ds(base * H, B1 * H)])
        return 0

    nblk = (NB1 - wid + NW - 1) // NW
    lax.fori_loop(0, nblk, lambda k, c: blk_body(wid + k * NW, c), 0)
    pltpu.sync_copy(acc_v, part_hbm.at[wid])


def _sc2_body(dst_hbm, ex_hbm, rd_hbm, sc_hbm,
              rd_v, dst_v, ex_v, sc_v):
    wid = lax.axis_index("s") * NC + lax.axis_index("c")
    pltpu.sync_copy(rd_hbm, rd_v)

    def blk_body(b, _):
        base = b * B1
        pltpu.sync_copy(dst_hbm.at[pl.ds(base, B1)], dst_v)
        pltpu.sync_copy(ex_hbm.at[pl.ds(base * H, B1 * H)], ex_v)

        def vec_body(i, _):
            d16 = dst_v[pl.ds(i * 16, 16)]
            lane = lax.iota(jnp.int32, 16) + i * 16
            for h in range(H):
                ex16 = plsc.load_gather(ex_v, [lane * H + h])
                r16 = plsc.load_gather(rd_v, [d16 * H + h])
                plsc.store_scatter(sc_v, [lane * H + h], ex16 * r16)
            return 0
        lax.fori_loop(0, B1 // 16, vec_body, 0)
        pltpu.sync_copy(sc_v, sc_hbm.at[pl.ds(base * H, B1 * H)])
        return 0

    nblk = (NB1 - wid + NW - 1) // NW
    lax.fori_loop(0, nblk, lambda k, c: blk_body(wid + k * NW, c), 0)


def _sc3_body(src_hbm, dst_hbm, wva_hbm, wvb_hbm, seea_hbm, seeb_hbm,
              agga_hbm, aggb_hbm,
              u0, u1, se0, se1, src0, src1, dst0, dst1,
              sem_u0, sem_u1, sem_s0, sem_s1, agg_s):
    c = lax.axis_index("c")
    s = lax.axis_index("s")
    u_v = (u0, u1)
    se_v = (se0, se1)
    src_v = (src0, src1)
    dst_v = (dst0, dst1)
    sem_u = (sem_u0, sem_u1)
    sem_s = (sem_s0, sem_s1)

    def zbuf_body(i, _):
        for j in range(QF // 16):
            u0[i, pl.ds(j * 16, 16)] = jnp.zeros((16,), jnp.float32)
        return 0
    lax.fori_loop(0, B3, zbuf_body, 0)
    off = 0
    for nrows in _ZCOPY:
        for rep in range(-(-nrows // B3)):
            nn = min(B3, nrows - rep * B3)
            pltpu.sync_copy(
                u0.at[pl.ds(0, nn)],
                agg_s.at[pl.ds(s * ROWS_PER_TILE + off + rep * B3, nn)])
        off += nrows

    @pl.when(s == NS - 1)
    def _():
        pltpu.sync_copy(u0.at[pl.ds(0, _TAIL)],
                        agg_s.at[pl.ds(NS * ROWS_PER_TILE, _TAIL)])

    plsc.subcore_barrier()

    nblk = (NB3 - s + NS - 1) // NS

    def issue(b, j):
        base = (s + j * NS) * B3
        pltpu.sync_copy(src_hbm.at[pl.ds(base, B3)], src_v[b])
        pltpu.sync_copy(dst_hbm.at[pl.ds(base, B3)], dst_v[b])

        @pl.when(c == 0)
        def _():
            pltpu.async_copy(wva_hbm.at[src_v[b]], u_v[b], sem_u[b])
            pltpu.async_copy(seea_hbm.at[pl.ds(base, B3)], se_v[b], sem_s[b])

        @pl.when(c == 1)
        def _():
            pltpu.async_copy(wvb_hbm.at[src_v[b]], u_v[b], sem_u[b])
            pltpu.async_copy(seeb_hbm.at[pl.ds(base, B3)], se_v[b], sem_s[b])

    for b in range(2):
        @pl.when(b < nblk)
        def _(b=b):
            issue(b, b)

    def k2_body(k2, _):
        for b in range(2):
            j = 2 * k2 + b

            @pl.when(j < nblk)
            def _(b=b, j=j):
                pltpu.make_async_copy(wva_hbm.at[src_v[b]], u_v[b],
                                      sem_u[b]).wait()
                pltpu.make_async_copy(seea_hbm.at[pl.ds(0, B3)], se_v[b],
                                      sem_s[b]).wait()

                def mul_body(i, _):
                    for jj in range(QF // 16):
                        u_v[b][i, pl.ds(jj * 16, 16)] = (
                            u_v[b][i, pl.ds(jj * 16, 16)]
                            * se_v[b][i, pl.ds(jj * 16, 16)])
                    return 0
                lax.fori_loop(0, B3, mul_body, 0)

                pltpu.sync_copy(u_v[b], agg_s.at[dst_v[b]], add=True)

                @pl.when(j + 2 < nblk)
                def _():
                    issue(b, j + 2)
        return 0

    lax.fori_loop(0, (nblk + 1) // 2, k2_body, 0)
    plsc.subcore_barrier()

    def _dump(row, nrows):
        @pl.when(c == 0)
        def _():
            pltpu.sync_copy(agg_s.at[pl.ds(row, nrows)],
                            agga_hbm.at[pl.ds(row, nrows)])

        @pl.when(c == 1)
        def _():
            pltpu.sync_copy(agg_s.at[pl.ds(row, nrows)],
                            aggb_hbm.at[pl.ds(row, nrows)])

    off = 0
    for nrows in _ZCOPY:
        _dump(s * ROWS_PER_TILE + off, nrows)
        off += nrows

    @pl.when(s == NS - 1)
    def _():
        _dump(NS * ROWS_PER_TILE, _TAIL)


@jax.jit
def kernel(h, edge_index, e, W_node, W_edge, w_att, W_scale, bias):
    f32 = jnp.float32
    src = edge_index[0].astype(jnp.int32)
    dst = edge_index[1].astype(jnp.int32)

    wa = w_att[0]
    wa_u, wa_e, wa_v = wa[:, :NF], wa[:, NF:2 * NF], wa[:, 2 * NF:]
    wn3 = W_node.reshape(NF, H, NF)
    wu = jnp.einsum("fhg,hg->fh", wn3, wa_u)
    wv_fold = jnp.einsum("fhg,hg->fh", wn3, wa_v)
    w_uv = jnp.concatenate([wu, wv_fold], axis=1)
    we3 = W_edge.reshape(e.shape[1], H, NF)
    wea = jnp.einsum("fhg,hg->fh", we3, wa_e)

    BN = 1000
    aa = pl.pallas_call(
        _tc_a1,
        grid=(N // BN,),
        in_specs=[
            pl.BlockSpec((BN, NF), lambda i: (i, 0)),
            pl.BlockSpec((NF, 2 * H), lambda i: (0, 0)),
        ],
        out_specs=pl.BlockSpec((BN, 2 * H), lambda i: (i, 0)),
        out_shape=jax.ShapeDtypeStruct((N, 2 * H), f32),
    )(h, w_uv)
    wv0, wv1, wv2, wv3 = pl.pallas_call(
        _tc_a2,
        grid=(N // BN,),
        in_specs=[
            pl.BlockSpec((BN, NF), lambda i: (i, 0)),
            pl.BlockSpec((NF, H * NF), lambda i: (0, 0)),
        ],
        out_specs=[pl.BlockSpec((BN, QF), lambda i: (i, 0))] * 4,
        out_shape=[jax.ShapeDtypeStruct((N, QF), f32)] * 4,
    )(h, W_node)

    BE = 2000
    ae = pl.pallas_call(
        _tc_b,
        grid=(E // BE,),
        in_specs=[
            pl.BlockSpec((BE, e.shape[1]), lambda i: (i, 0)),
            pl.BlockSpec((e.shape[1], H), lambda i: (0, 0)),
        ],
        out_specs=pl.BlockSpec((BE, H), lambda i: (i, 0)),
        out_shape=jax.ShapeDtypeStruct((E, H), f32),
    )(e, wea)

    mesh = plsc.VectorSubcoreMesh(core_axis_name="c", subcore_axis_name="s")
    sc_params = pltpu.CompilerParams(needs_layout_passes=False,
                                     use_tc_tiling_on_sc=False)

    sc1 = pl.kernel(
        _sc1_body,
        out_type=[
            jax.ShapeDtypeStruct((E * H,), f32),
            jax.ShapeDtypeStruct((NW, N * H), f32),
        ],
        mesh=mesh,
        scratch_types=[
            pltpu.VMEM((N * 2 * H,), f32),
            pltpu.VMEM((N * H,), f32),
            pltpu.VMEM((B1,), jnp.int32),
            pltpu.VMEM((B1,), jnp.int32),
            pltpu.VMEM((B1 * H,), f32),
            pltpu.VMEM((B1 * H,), f32),
        ],
        compiler_params=sc_params,
    )
    ex, part = sc1(src, dst, ae.reshape(E * H), aa.reshape(N * 2 * H))

    rd = pl.pallas_call(
        _tc_c,
        grid=(1,),
        in_specs=[pl.BlockSpec((NW, N * H), lambda i: (0, 0))],
        out_specs=pl.BlockSpec((1, N * H), lambda i: (0, 0)),
        out_shape=jax.ShapeDtypeStruct((1, N * H), f32),
    )(part)

    sc2 = pl.kernel(
        _sc2_body,
        out_type=jax.ShapeDtypeStruct((E * H,), f32),
        mesh=mesh,
        scratch_types=[
            pltpu.VMEM((N * H,), f32),
            pltpu.VMEM((B1,), jnp.int32),
            pltpu.VMEM((B1 * H,), f32),
            pltpu.VMEM((B1 * H,), f32),
        ],
        compiler_params=sc_params,
    )
    score = sc2(dst, ex, rd.reshape(N * H))

    BD = 1000
    see_q = pl.pallas_call(
        _tc_d,
        grid=(E // BD,),
        in_specs=[
            pl.BlockSpec((BD, e.shape[1]), lambda i: (i, 0)),
            pl.BlockSpec((BD, H), lambda i: (i, 0)),
            pl.BlockSpec((e.shape[1], H * NF), lambda i: (0, 0)),
        ],
        out_specs=[pl.BlockSpec((BD, QF), lambda i: (i, 0))] * 4,
        out_shape=[jax.ShapeDtypeStruct((E, QF), f32)] * 4,
    )(e, score.reshape(E, H), W_edge)

    sc3 = pl.kernel(
        _sc3_body,
        out_type=[jax.ShapeDtypeStruct((N, QF), f32)] * 2,
        mesh=mesh,
        scratch_types=(
            [pltpu.VMEM((B3, QF), f32)] * 4
            + [pltpu.VMEM((B3,), jnp.int32)] * 4
            + [pltpu.SemaphoreType.DMA] * 4
            + [pltpu.VMEM_SHARED((N, QF), f32)]
        ),
        compiler_params=sc_params,
    )
    agg0, agg1 = sc3(src, dst, wv0, wv1, see_q[0], see_q[1])
    agg2, agg3 = sc3(src, dst, wv2, wv3, see_q[2], see_q[3])

    out = pl.pallas_call(
        _tc_e,
        grid=(N // BN,),
        in_specs=[pl.BlockSpec((BN, QF), lambda i: (i, 0))] * 4
        + [
            pl.BlockSpec((H * NF, NF), lambda i: (0, 0)),
            pl.BlockSpec((1, NF), lambda i: (0, 0)),
        ],
        out_specs=pl.BlockSpec((BN, NF), lambda i: (i, 0)),
        out_shape=jax.ShapeDtypeStruct((N, NF), f32),
    )(agg0, agg1, agg2, agg3, W_scale, bias.reshape(1, NF))

    return out
